# baseline (speedup 1.0000x reference)
import jax
import jax.numpy as jnp
from jax import lax
from jax.experimental import pallas as pl
from jax.experimental.pallas import tpu as pltpu
from jax.experimental.pallas import tpu_sc as plsc

G = 512
N = 1048576
NW = 32
K = 1024
NG = K // 16
NCHUNK = N // NW // K
W = 40

RPW = G * G // NW
LC = 1024
NBC = RPW // LC
SROWS = 19
SCELLS = SROWS * G

_A = ((-0.5, 1.5, -1.5, 0.5),
      (1.0, -2.5, 2.0, -0.5),
      (-0.5, 0.0, 0.5, 0.0),
      (0.0, 1.0, 0.0, 0.0))

_params = pltpu.CompilerParams(
    needs_layout_passes=False, use_tc_tiling_on_sc=False,
    disable_bounds_checks=True)
_mesh = lambda: plsc.VectorSubcoreMesh(core_axis_name="c", subcore_axis_name="s")


def _basis(t):
    ws = []
    for i in range(4):
        w = jnp.full((16,), _A[0][i], jnp.float32)
        for a in range(1, 4):
            w = w * t + jnp.float32(_A[a][i])
        ws.append(w)
    return ws


def _build_body(cp0_hbm, cp1_hbm, p_hbm, slab0, slab1, pbuf,
                sstage, sout0, sout1):
    wid = lax.axis_index("s") * 2 + lax.axis_index("c")
    iota = lax.iota(jnp.int32, 16)
    souts = (sout0, sout1)
    slabs = (slab0, slab1)

    gw0 = wid * RPW
    sr = jnp.minimum(jnp.maximum(wid * 16 - 1, 0), G - SROWS)
    srcell = sr * G

    pltpu.async_copy(cp0_hbm.at[pl.ds(srcell, SCELLS)], slab0, sstage)
    pltpu.async_copy(cp1_hbm.at[pl.ds(srcell, SCELLS)], slab1, sstage)
    pltpu.make_async_copy(cp0_hbm.at[pl.ds(0, SCELLS)], slab0, sstage).wait()
    pltpu.make_async_copy(cp1_hbm.at[pl.ds(0, SCELLS)], slab1, sstage).wait()

    def chunk(ci, b):
        g0 = gw0 + ci * LC
        pslot = pbuf.at[b]

        def grp(gi, carry):
            gv = g0 + gi * 16 + iota
            bv = gv - srcell
            rows = gi * 16 + iota
            for i in range(4):
                for j in range(4):
                    cell = bv + ((i - 1) * G + j - 1)
                    cell = jnp.minimum(jnp.maximum(cell, 0), SCELLS - 1)
                    for d in range(2):
                        k = 8 * i + 2 * j + d
                        val = plsc.load_gather(slabs[d], [cell])
                        plsc.store_scatter(pslot, [rows, iota * 0 + k], val)
            return carry
        lax.fori_loop(0, LC // 16, grp, 0, unroll=False)
        pltpu.async_copy(pslot, p_hbm.at[pl.ds(g0, LC), :], souts[b])

    def wait_out(b):
        pltpu.make_async_copy(pbuf.at[b], p_hbm.at[pl.ds(0, LC), :],
                              souts[b]).wait()

    for ci in range(NBC):
        b = ci % 2
        if ci >= 2:
            wait_out(b)
        chunk(ci, b)
    wait_out(NBC % 2)
    wait_out(1 - NBC % 2)


def _main_body(x_hbm, y_hbm, u_hbm, v_hbm, r_hbm, c_hbm, tab_hbm, out_hbm,
               xc, yc, uc, vc, rc, cc, gidx, qbuf, accv,
               sin0, sin1, sg0, sg1):
    wid = lax.axis_index("s") * 2 + lax.axis_index("c")
    base_pt = wid * (N // NW)
    iota = lax.iota(jnp.int32, 16)
    zf16 = jnp.zeros((16,), jnp.float32)
    sins = (sin0, sin1)
    sgs = (sg0, sg1)
    planes = ((x_hbm, xc), (y_hbm, yc), (u_hbm, uc), (v_hbm, vc),
              (r_hbm, rc), (c_hbm, cc))

    def stage(ch, b):
        off = base_pt + ch * K
        for hbm, buf in planes:
            pltpu.async_copy(hbm.at[pl.ds(off, K)], buf.at[b], sins[b])

    def wait_in(b):
        for hbm, buf in planes:
            pltpu.make_async_copy(hbm.at[pl.ds(0, K)], buf.at[b],
                                  sins[b]).wait()

    def build(b):
        rslot = rc.at[b]
        cslot = cc.at[b]
        gslot = gidx.at[b]

        def body(g, carry):
            o = g * 16
            r = rslot[pl.ds(o, 16)]
            c = cslot[pl.ds(o, 16)]
            gslot[pl.ds(o, 16)] = r * G + c
            return carry
        lax.fori_loop(0, NG, body, 0, unroll=False)

    def fire_gather(b):
        pltpu.async_copy(tab_hbm.at[gidx.at[b]], qbuf.at[b], sgs[b])

    def wait_gather(b):
        pltpu.make_async_copy(tab_hbm.at[gidx.at[b]], qbuf.at[b], sgs[b]).wait()

    def compute(b, acc):
        qslot = qbuf.at[b]

        def body(g, acc):
            o = g * 16
            nv = o + iota
            x = xc[b, pl.ds(o, 16)]
            y = yc[b, pl.ds(o, 16)]
            u = uc[b, pl.ds(o, 16)]
            v = vc[b, pl.ds(o, 16)]
            x = x - x.astype(jnp.int32).astype(jnp.float32)
            y = y - y.astype(jnp.int32).astype(jnp.float32)
            wx = _basis(x)
            wy = _basis(y)
            ms = []
            for dch in range(2):
                terms = []
                for i in range(4):
                    qs = []
                    for j in range(4):
                        col = 8 * i + 2 * j + dch
                        qs.append(plsc.load_gather(qslot,
                                                   [nv, iota * 0 + col]))
                    s = ((wy[0] * qs[0] + wy[1] * qs[1])
                         + (wy[2] * qs[2] + wy[3] * qs[3]))
                    terms.append(wx[i] * s)
                ms.append((terms[0] + terms[1]) + (terms[2] + terms[3]))
            d0 = u - ms[0]
            d1 = v - ms[1]
            return acc + (d0 * d0 + d1 * d1)
        return lax.fori_loop(0, NG, body, acc, unroll=2)

    stage(0, 0)
    wait_in(0)
    build(0)
    fire_gather(0)
    stage(1, 1)

    def outer(o, acc):
        for b in (0, 1):
            ch = 2 * o + b
            nb = 1 - b

            @pl.when(ch + 1 < NCHUNK)
            def _():
                wait_in(nb)
                build(nb)
                fire_gather(nb)

            wait_gather(b)
            acc = compute(b, acc)

            @pl.when(ch + 2 < NCHUNK)
            def _():
                stage(ch + 2, b)
        return acc

    acc = lax.fori_loop(0, NCHUNK // 2, outer,
                        jnp.zeros((16,), jnp.float32), unroll=False)
    accv[...] = acc
    pltpu.sync_copy(accv, out_hbm.at[wid])


def kernel(ch1, ch2, CP_locs, CP_idx):
    cp0 = CP_locs[:, :, 0].reshape(G * G)
    cp1 = CP_locs[:, :, 1].reshape(G * G)
    x = ch2[:, 0]
    y = ch2[:, 1]
    u = ch1[:, 0]
    v = ch1[:, 1]
    r = CP_idx[:, 0]
    c = CP_idx[:, 1]

    ptab = pl.kernel(
        _build_body,
        out_type=jax.ShapeDtypeStruct((G * G, W), jnp.float32),
        mesh=_mesh(),
        compiler_params=_params,
        scratch_types=[
            pltpu.VMEM((SCELLS,), jnp.float32),
            pltpu.VMEM((SCELLS,), jnp.float32),
            pltpu.VMEM((2, LC, W), jnp.float32),
            pltpu.SemaphoreType.DMA,
            pltpu.SemaphoreType.DMA,
            pltpu.SemaphoreType.DMA,
        ],
    )(cp0, cp1)

    partials = pl.kernel(
        _main_body,
        out_type=jax.ShapeDtypeStruct((NW, 16), jnp.float32),
        mesh=_mesh(),
        compiler_params=_params,
        scratch_types=[
            pltpu.VMEM((2, K), jnp.float32),
            pltpu.VMEM((2, K), jnp.float32),
            pltpu.VMEM((2, K), jnp.float32),
            pltpu.VMEM((2, K), jnp.float32),
            pltpu.VMEM((2, K), jnp.int32),
            pltpu.VMEM((2, K), jnp.int32),
            pltpu.VMEM((2, K), jnp.int32),
            pltpu.VMEM((2, K, W), jnp.float32),
            pltpu.VMEM((16,), jnp.float32),
            pltpu.SemaphoreType.DMA,
            pltpu.SemaphoreType.DMA,
            pltpu.SemaphoreType.DMA,
            pltpu.SemaphoreType.DMA,
        ],
    )(x, y, u, v, r, c, ptab)
    return jnp.sum(partials)


# speedup vs baseline: 1.5245x; 1.5245x over previous
import jax
import jax.numpy as jnp
from jax import lax
from jax.experimental import pallas as pl
from jax.experimental.pallas import tpu as pltpu
from jax.experimental.pallas import tpu_sc as plsc

G = 512
N = 1048576
NW = 32
K = 1024
NG = K // 16
NCHUNK = N // NW // K
W = 40

RPW = G * G // NW
LC = 1024
NBC = RPW // LC
SROWS = 19
SCELLS = SROWS * G

_A = ((-0.5, 1.5, -1.5, 0.5),
      (1.0, -2.5, 2.0, -0.5),
      (-0.5, 0.0, 0.5, 0.0),
      (0.0, 1.0, 0.0, 0.0))

_params = pltpu.CompilerParams(
    needs_layout_passes=False, use_tc_tiling_on_sc=False,
    disable_bounds_checks=True)
_mesh = lambda: plsc.VectorSubcoreMesh(core_axis_name="c", subcore_axis_name="s")


def _basis(t):
    ws = []
    for i in range(4):
        w = jnp.full((16,), _A[0][i], jnp.float32)
        for a in range(1, 4):
            w = w * t + jnp.float32(_A[a][i])
        ws.append(w)
    return ws


def _build_body(cp0_hbm, cp1_hbm, p_hbm, slab0, slab1, pbuf,
                sstage, sout0, sout1):
    wid = lax.axis_index("s") * 2 + lax.axis_index("c")
    iota = lax.iota(jnp.int32, 16)
    souts = (sout0, sout1)
    slabs = (slab0, slab1)

    gw0 = wid * RPW
    sr = jnp.minimum(jnp.maximum(wid * 16 - 1, 0), G - SROWS)
    srcell = sr * G

    pltpu.async_copy(cp0_hbm.at[pl.ds(srcell, SCELLS)], slab0, sstage)
    pltpu.async_copy(cp1_hbm.at[pl.ds(srcell, SCELLS)], slab1, sstage)
    pltpu.make_async_copy(cp0_hbm.at[pl.ds(0, SCELLS)], slab0, sstage).wait()
    pltpu.make_async_copy(cp1_hbm.at[pl.ds(0, SCELLS)], slab1, sstage).wait()

    def chunk(ci, b):
        g0 = gw0 + ci * LC
        pslot = pbuf.at[b]

        def grp(gi, carry):
            gv = g0 + gi * 16 + iota
            bv = gv - srcell
            rows = gi * 16 + iota
            for i in range(4):
                for j in range(4):
                    cell = bv + ((i - 1) * G + j - 1)
                    cell = jnp.minimum(jnp.maximum(cell, 0), SCELLS - 1)
                    for d in range(2):
                        k = 8 * i + 2 * j + d
                        val = plsc.load_gather(slabs[d], [cell])
                        plsc.store_scatter(pslot, [rows, iota * 0 + k], val)
            return carry
        lax.fori_loop(0, LC // 16, grp, 0, unroll=False)
        pltpu.async_copy(pslot, p_hbm.at[pl.ds(g0, LC), :], souts[b])

    def wait_out(b):
        pltpu.make_async_copy(pbuf.at[b], p_hbm.at[pl.ds(0, LC), :],
                              souts[b]).wait()

    for ci in range(NBC):
        b = ci % 2
        if ci >= 2:
            wait_out(b)
        chunk(ci, b)
    wait_out(NBC % 2)
    wait_out(1 - NBC % 2)


def _main_body(x_hbm, y_hbm, u_hbm, v_hbm, r_hbm, c_hbm, tab_hbm, out_hbm,
               xc, yc, uc, vc, rc, cc, gidx, qbuf, accv,
               sin0, sin1, sg0, sg1):
    wid = lax.axis_index("s") * 2 + lax.axis_index("c")
    base_pt = wid * (N // NW)
    iota = lax.iota(jnp.int32, 16)
    zf16 = jnp.zeros((16,), jnp.float32)
    sins = (sin0, sin1)
    sgs = (sg0, sg1)
    planes = ((x_hbm, xc), (y_hbm, yc), (u_hbm, uc), (v_hbm, vc),
              (r_hbm, rc), (c_hbm, cc))

    def stage(ch, b):
        off = base_pt + ch * K
        for hbm, buf in planes:
            pltpu.async_copy(hbm.at[pl.ds(off, K)], buf.at[b], sins[b])

    def wait_in(b):
        for hbm, buf in planes:
            pltpu.make_async_copy(hbm.at[pl.ds(0, K)], buf.at[b],
                                  sins[b]).wait()

    def build(b):
        rslot = rc.at[b]
        cslot = cc.at[b]
        gslot = gidx.at[b]

        def body(g, carry):
            o = g * 16
            r = rslot[pl.ds(o, 16)]
            c = cslot[pl.ds(o, 16)]
            gslot[pl.ds(o, 16)] = r * G + c
            return carry
        lax.fori_loop(0, NG, body, 0, unroll=False)

    def fire_gather(b):
        pltpu.async_copy(tab_hbm.at[gidx.at[b]], qbuf.at[b], sgs[b])

    def wait_gather(b):
        pltpu.make_async_copy(tab_hbm.at[gidx.at[b]], qbuf.at[b], sgs[b]).wait()

    def compute(b, acc):
        qslot = qbuf.at[b]

        def body(g, acc):
            o = g * 16
            nv = o + iota
            x = xc[b, pl.ds(o, 16)]
            y = yc[b, pl.ds(o, 16)]
            u = uc[b, pl.ds(o, 16)]
            v = vc[b, pl.ds(o, 16)]
            x = x - x.astype(jnp.int32).astype(jnp.float32)
            y = y - y.astype(jnp.int32).astype(jnp.float32)
            wx = _basis(x)
            wy = _basis(y)
            ms = []
            for dch in range(2):
                terms = []
                for i in range(4):
                    qs = []
                    for j in range(4):
                        col = 8 * i + 2 * j + dch
                        qs.append(plsc.load_gather(qslot,
                                                   [nv, iota * 0 + col]))
                    s = ((wy[0] * qs[0] + wy[1] * qs[1])
                         + (wy[2] * qs[2] + wy[3] * qs[3]))
                    terms.append(wx[i] * s)
                ms.append((terms[0] + terms[1]) + (terms[2] + terms[3]))
            d0 = u - ms[0]
            d1 = v - ms[1]
            return acc + (d0 * d0 + d1 * d1)
        return lax.fori_loop(0, NG, body, acc, unroll=False)

    stage(0, 0)
    wait_in(0)
    build(0)
    fire_gather(0)
    stage(1, 1)

    def outer(o, acc):
        for b in (0, 1):
            ch = 2 * o + b
            nb = 1 - b

            @pl.when(ch + 1 < NCHUNK)
            def _():
                wait_in(nb)
                build(nb)
                fire_gather(nb)

            wait_gather(b)
            acc = compute(b, acc)

            @pl.when(ch + 2 < NCHUNK)
            def _():
                stage(ch + 2, b)
        return acc

    acc = lax.fori_loop(0, NCHUNK // 2, outer,
                        jnp.zeros((16,), jnp.float32), unroll=False)
    accv[...] = acc
    pltpu.sync_copy(accv, out_hbm.at[wid])


def kernel(ch1, ch2, CP_locs, CP_idx):
    cp0 = CP_locs[:, :, 0].reshape(G * G)
    cp1 = CP_locs[:, :, 1].reshape(G * G)
    x = ch2[:, 0]
    y = ch2[:, 1]
    u = ch1[:, 0]
    v = ch1[:, 1]
    r = CP_idx[:, 0]
    c = CP_idx[:, 1]

    ptab = pl.kernel(
        _build_body,
        out_type=jax.ShapeDtypeStruct((G * G, W), jnp.float32),
        mesh=_mesh(),
        compiler_params=_params,
        scratch_types=[
            pltpu.VMEM((SCELLS,), jnp.float32),
            pltpu.VMEM((SCELLS,), jnp.float32),
            pltpu.VMEM((2, LC, W), jnp.float32),
            pltpu.SemaphoreType.DMA,
            pltpu.SemaphoreType.DMA,
            pltpu.SemaphoreType.DMA,
        ],
    )(cp0, cp1)

    partials = pl.kernel(
        _main_body,
        out_type=jax.ShapeDtypeStruct((NW, 16), jnp.float32),
        mesh=_mesh(),
        compiler_params=_params,
        scratch_types=[
            pltpu.VMEM((2, K), jnp.float32),
            pltpu.VMEM((2, K), jnp.float32),
            pltpu.VMEM((2, K), jnp.float32),
            pltpu.VMEM((2, K), jnp.float32),
            pltpu.VMEM((2, K), jnp.int32),
            pltpu.VMEM((2, K), jnp.int32),
            pltpu.VMEM((2, K), jnp.int32),
            pltpu.VMEM((2, K, W), jnp.float32),
            pltpu.VMEM((16,), jnp.float32),
            pltpu.SemaphoreType.DMA,
            pltpu.SemaphoreType.DMA,
            pltpu.SemaphoreType.DMA,
            pltpu.SemaphoreType.DMA,
        ],
    )(x, y, u, v, r, c, ptab)
    return jnp.sum(partials)


# speedup vs baseline: 1.5899x; 1.0429x over previous
import jax
import jax.numpy as jnp
from jax import lax
from jax.experimental import pallas as pl
from jax.experimental.pallas import tpu as pltpu
from jax.experimental.pallas import tpu_sc as plsc

G = 512
N = 1048576
NW = 32
K = 1024
NG = K // 16
NCHUNK = N // NW // K
W = 40

RPW = G * G // NW
LC = 1024
NBC = RPW // LC
SROWS = 19
SCELLS = SROWS * G

_A = ((-0.5, 1.5, -1.5, 0.5),
      (1.0, -2.5, 2.0, -0.5),
      (-0.5, 0.0, 0.5, 0.0),
      (0.0, 1.0, 0.0, 0.0))

_params = pltpu.CompilerParams(
    needs_layout_passes=False, use_tc_tiling_on_sc=False,
    disable_bounds_checks=True)
_mesh = lambda: plsc.VectorSubcoreMesh(core_axis_name="c", subcore_axis_name="s")


def _basis(t):
    ws = []
    for i in range(4):
        w = jnp.full((16,), _A[0][i], jnp.float32)
        for a in range(1, 4):
            w = w * t + jnp.float32(_A[a][i])
        ws.append(w)
    return ws


def _build_body(cp0_hbm, cp1_hbm, p_hbm, slab0, slab1, pbuf,
                sstage, sout0, sout1):
    wid = lax.axis_index("s") * 2 + lax.axis_index("c")
    iota = lax.iota(jnp.int32, 16)
    souts = (sout0, sout1)
    slabs = (slab0, slab1)

    gw0 = wid * RPW
    sr = jnp.minimum(jnp.maximum(wid * 16 - 1, 0), G - SROWS)
    srcell = sr * G

    pltpu.async_copy(cp0_hbm.at[pl.ds(srcell, SCELLS)],
                     slab0.at[pl.ds(16, SCELLS)], sstage)
    pltpu.async_copy(cp1_hbm.at[pl.ds(srcell, SCELLS)],
                     slab1.at[pl.ds(16, SCELLS)], sstage)
    pltpu.make_async_copy(cp0_hbm.at[pl.ds(0, SCELLS)],
                          slab0.at[pl.ds(16, SCELLS)], sstage).wait()
    pltpu.make_async_copy(cp1_hbm.at[pl.ds(0, SCELLS)],
                          slab1.at[pl.ds(16, SCELLS)], sstage).wait()

    def chunk(ci, b):
        g0 = gw0 + ci * LC
        pslot = pbuf.at[b]

        def grp(gi, carry):
            base = g0 + gi * 16 - srcell
            rows = gi * 16 + iota
            for i in range(4):
                for j in range(4):
                    start = base + (16 + (i - 1) * G + j - 1)
                    start = jnp.maximum(start, 0)
                    for d in range(2):
                        k = 8 * i + 2 * j + d
                        val = slabs[d][pl.ds(start, 16)]
                        plsc.store_scatter(pslot, [rows, iota * 0 + k], val)
            return carry
        lax.fori_loop(0, LC // 16, grp, 0, unroll=False)
        pltpu.async_copy(pslot, p_hbm.at[pl.ds(g0, LC), :], souts[b])

    def wait_out(b):
        pltpu.make_async_copy(pbuf.at[b], p_hbm.at[pl.ds(0, LC), :],
                              souts[b]).wait()

    for ci in range(NBC):
        b = ci % 2
        if ci >= 2:
            wait_out(b)
        chunk(ci, b)
    wait_out(NBC % 2)
    wait_out(1 - NBC % 2)


def _main_body(x_hbm, y_hbm, u_hbm, v_hbm, r_hbm, c_hbm, tab_hbm, out_hbm,
               xc, yc, uc, vc, rc, cc, gidx, qbuf, accv,
               sin0, sin1, sg0, sg1):
    wid = lax.axis_index("s") * 2 + lax.axis_index("c")
    base_pt = wid * (N // NW)
    iota = lax.iota(jnp.int32, 16)
    zf16 = jnp.zeros((16,), jnp.float32)
    sins = (sin0, sin1)
    sgs = (sg0, sg1)
    planes = ((x_hbm, xc), (y_hbm, yc), (u_hbm, uc), (v_hbm, vc),
              (r_hbm, rc), (c_hbm, cc))

    def stage(ch, b):
        off = base_pt + ch * K
        for hbm, buf in planes:
            pltpu.async_copy(hbm.at[pl.ds(off, K)], buf.at[b], sins[b])

    def wait_in(b):
        for hbm, buf in planes:
            pltpu.make_async_copy(hbm.at[pl.ds(0, K)], buf.at[b],
                                  sins[b]).wait()

    def build(b):
        rslot = rc.at[b]
        cslot = cc.at[b]
        gslot = gidx.at[b]

        def body(g, carry):
            o = g * 16
            r = rslot[pl.ds(o, 16)]
            c = cslot[pl.ds(o, 16)]
            gslot[pl.ds(o, 16)] = r * G + c
            return carry
        lax.fori_loop(0, NG, body, 0, unroll=False)

    def fire_gather(b):
        pltpu.async_copy(tab_hbm.at[gidx.at[b]], qbuf.at[b], sgs[b])

    def wait_gather(b):
        pltpu.make_async_copy(tab_hbm.at[gidx.at[b]], qbuf.at[b], sgs[b]).wait()

    def compute(b, acc):
        qslot = qbuf.at[b]

        def body(g, acc):
            o = g * 16
            nv = o + iota
            x = xc[b, pl.ds(o, 16)]
            y = yc[b, pl.ds(o, 16)]
            u = uc[b, pl.ds(o, 16)]
            v = vc[b, pl.ds(o, 16)]
            x = x - x.astype(jnp.int32).astype(jnp.float32)
            y = y - y.astype(jnp.int32).astype(jnp.float32)
            wx = _basis(x)
            wy = _basis(y)
            ms = []
            for dch in range(2):
                terms = []
                for i in range(4):
                    qs = []
                    for j in range(4):
                        col = 8 * i + 2 * j + dch
                        qs.append(plsc.load_gather(qslot,
                                                   [nv, iota * 0 + col]))
                    s = ((wy[0] * qs[0] + wy[1] * qs[1])
                         + (wy[2] * qs[2] + wy[3] * qs[3]))
                    terms.append(wx[i] * s)
                ms.append((terms[0] + terms[1]) + (terms[2] + terms[3]))
            d0 = u - ms[0]
            d1 = v - ms[1]
            return acc + (d0 * d0 + d1 * d1)
        return lax.fori_loop(0, NG, body, acc, unroll=False)

    stage(0, 0)
    wait_in(0)
    build(0)
    fire_gather(0)
    stage(1, 1)

    def outer(o, acc):
        for b in (0, 1):
            ch = 2 * o + b
            nb = 1 - b

            @pl.when(ch + 1 < NCHUNK)
            def _():
                wait_in(nb)
                build(nb)
                fire_gather(nb)

            wait_gather(b)
            acc = compute(b, acc)

            @pl.when(ch + 2 < NCHUNK)
            def _():
                stage(ch + 2, b)
        return acc

    acc = lax.fori_loop(0, NCHUNK // 2, outer,
                        jnp.zeros((16,), jnp.float32), unroll=False)
    accv[...] = acc
    pltpu.sync_copy(accv, out_hbm.at[wid])


def kernel(ch1, ch2, CP_locs, CP_idx):
    cp0 = CP_locs[:, :, 0].reshape(G * G)
    cp1 = CP_locs[:, :, 1].reshape(G * G)
    x = ch2[:, 0]
    y = ch2[:, 1]
    u = ch1[:, 0]
    v = ch1[:, 1]
    r = CP_idx[:, 0]
    c = CP_idx[:, 1]

    ptab = pl.kernel(
        _build_body,
        out_type=jax.ShapeDtypeStruct((G * G, W), jnp.float32),
        mesh=_mesh(),
        compiler_params=_params,
        scratch_types=[
            pltpu.VMEM((SCELLS + 32,), jnp.float32),
            pltpu.VMEM((SCELLS + 32,), jnp.float32),
            pltpu.VMEM((2, LC, W), jnp.float32),
            pltpu.SemaphoreType.DMA,
            pltpu.SemaphoreType.DMA,
            pltpu.SemaphoreType.DMA,
        ],
    )(cp0, cp1)

    partials = pl.kernel(
        _main_body,
        out_type=jax.ShapeDtypeStruct((NW, 16), jnp.float32),
        mesh=_mesh(),
        compiler_params=_params,
        scratch_types=[
            pltpu.VMEM((2, K), jnp.float32),
            pltpu.VMEM((2, K), jnp.float32),
            pltpu.VMEM((2, K), jnp.float32),
            pltpu.VMEM((2, K), jnp.float32),
            pltpu.VMEM((2, K), jnp.int32),
            pltpu.VMEM((2, K), jnp.int32),
            pltpu.VMEM((2, K), jnp.int32),
            pltpu.VMEM((2, K, W), jnp.float32),
            pltpu.VMEM((16,), jnp.float32),
            pltpu.SemaphoreType.DMA,
            pltpu.SemaphoreType.DMA,
            pltpu.SemaphoreType.DMA,
            pltpu.SemaphoreType.DMA,
        ],
    )(x, y, u, v, r, c, ptab)
    return jnp.sum(partials)
